# baseline (speedup 1.0000x reference)
import jax
import jax.numpy as jnp
from jax.experimental import pallas as pl
from jax.experimental.pallas import tpu as pltpu

E = 8
TOP_K = 2
D = 2048
D_FF = 1024
SHARED_FF = 2048
RSF = 2.5
T = 2048

CH = 128
SC_CH = SHARED_FF // CH
PE = D_FF // CH
NC = SC_CH + E * PE

_BF = jnp.bfloat16
_F32 = jnp.float32


def _silu(x):
    return x * jax.nn.sigmoid(x)


def _gate_body(x_ref, wgate_ref, w_ref):
    logits = jax.lax.dot_general(
        x_ref[...], wgate_ref[...], (((1,), (1,)), ((), ())),
        preferred_element_type=_F32)
    s = jax.nn.sigmoid(logits)
    lane = jax.lax.broadcasted_iota(jnp.int32, (T, E), 1)
    rank = jnp.zeros((T, E), jnp.int32)
    for j in range(E):
        sj = s[:, j:j + 1]
        rank += ((sj > s) | ((sj == s) & (j < lane))).astype(jnp.int32)
    w = jnp.where(rank < TOP_K, s, 0.0)
    denom = jnp.sum(w, axis=1, keepdims=True) + 1e-20
    w_ref[...] = w * (RSF / denom)


def _moe_body(xbf_ref, w_ref, sg_ref, su_ref, sd_ref,
              eg_ref, eu_ref, ed_ref, out_ref):
    c = pl.program_id(0)
    shared_phase = c < SC_CH
    xb = xbf_ref[...]

    wg = jnp.where(shared_phase, sg_ref[...], eg_ref[...].reshape(D, CH))
    wu = jnp.where(shared_phase, su_ref[...], eu_ref[...].reshape(D, CH))
    wd = jnp.where(shared_phase, sd_ref[...], ed_ref[...].reshape(CH, D))

    g = jnp.dot(xb, wg.astype(_BF), preferred_element_type=_F32)
    u = jnp.dot(xb, wu.astype(_BF), preferred_element_type=_F32)
    p = _silu(g) * u

    e_idx = (c - SC_CH) // PE
    lane = jax.lax.broadcasted_iota(jnp.int32, (T, E), 1)
    wcol = jnp.sum(jnp.where(lane == e_idx, w_ref[...], 0.0), axis=1,
                   keepdims=True)
    wcol = jnp.where(shared_phase, 1.0, wcol)
    pb = (p * wcol).astype(_BF)

    @pl.when(c == 0)
    def _():
        out_ref[...] = jnp.zeros((T, D), _F32)

    wdb = wd.astype(_BF)
    for h in range(2):
        lo, hi = h * (D // 2), (h + 1) * (D // 2)
        out_ref[:, lo:hi] += jnp.dot(pb, wdb[:, lo:hi],
                                     preferred_element_type=_F32)


def kernel(hidden_states, W_gate, Ws_gate, Ws_up, Ws_down, We_gate, We_up,
           We_down):
    B, S, Dm = hidden_states.shape
    x = hidden_states.reshape(T, D)
    xbf = x.astype(_BF)

    w = pl.pallas_call(
        _gate_body,
        in_specs=[
            pl.BlockSpec((T, D), lambda: (0, 0)),
            pl.BlockSpec((E, D), lambda: (0, 0)),
        ],
        out_specs=pl.BlockSpec((T, E), lambda: (0, 0)),
        out_shape=jax.ShapeDtypeStruct((T, E), _F32),
    )(x, W_gate)

    def _cr(c):
        return jnp.maximum(c - SC_CH, 0)

    out = pl.pallas_call(
        _moe_body,
        grid=(NC,),
        in_specs=[
            pl.BlockSpec((T, D), lambda c: (0, 0)),
            pl.BlockSpec((T, E), lambda c: (0, 0)),
            pl.BlockSpec((D, CH), lambda c: (0, jnp.minimum(c, SC_CH - 1))),
            pl.BlockSpec((D, CH), lambda c: (0, jnp.minimum(c, SC_CH - 1))),
            pl.BlockSpec((CH, D), lambda c: (jnp.minimum(c, SC_CH - 1), 0)),
            pl.BlockSpec((1, D, CH), lambda c: (_cr(c) // PE, 0, _cr(c) % PE)),
            pl.BlockSpec((1, D, CH), lambda c: (_cr(c) // PE, 0, _cr(c) % PE)),
            pl.BlockSpec((1, CH, D), lambda c: (_cr(c) // PE, _cr(c) % PE, 0)),
        ],
        out_specs=pl.BlockSpec((T, D), lambda c: (0, 0)),
        out_shape=jax.ShapeDtypeStruct((T, D), _F32),
    )(xbf, w, Ws_gate, Ws_up, Ws_down, We_gate, We_up, We_down)

    return out.reshape(B, S, Dm)


# speedup vs baseline: 1.7501x; 1.7501x over previous
import jax
import jax.numpy as jnp
from jax.experimental import pallas as pl
from jax.experimental.pallas import tpu as pltpu

E = 8
TOP_K = 2
D = 2048
D_FF = 1024
SHARED_FF = 2048
RSF = 2.5
T = 2048

CH = 256
SC_CH = SHARED_FF // CH
PE = D_FF // CH
NC = SC_CH + E * PE

_BF = jnp.bfloat16
_F32 = jnp.float32


def _silu(x):
    return x * jax.nn.sigmoid(x)


def _gate_body(x_ref, wgate_ref, w_ref):
    logits = jax.lax.dot_general(
        x_ref[...], wgate_ref[...], (((1,), (1,)), ((), ())),
        preferred_element_type=_F32)
    s = jax.nn.sigmoid(logits)
    lane = jax.lax.broadcasted_iota(jnp.int32, (T, E), 1)
    rank = jnp.zeros((T, E), jnp.int32)
    for j in range(E):
        sj = s[:, j:j + 1]
        rank += ((sj > s) | ((sj == s) & (j < lane))).astype(jnp.int32)
    w = jnp.where(rank < TOP_K, s, 0.0)
    denom = jnp.sum(w, axis=1, keepdims=True) + 1e-20
    w_ref[...] = w * (RSF / denom)


def _moe_body(xbf_ref, w_ref, sg_ref, su_ref, sd_ref,
              eg_ref, eu_ref, ed_ref, out_ref):
    c = pl.program_id(0)
    shared_phase = c < SC_CH
    xb = xbf_ref[...]

    wg = jnp.where(shared_phase, sg_ref[...], eg_ref[...].reshape(D, CH))
    wu = jnp.where(shared_phase, su_ref[...], eu_ref[...].reshape(D, CH))
    wd = jnp.where(shared_phase, sd_ref[...], ed_ref[...].reshape(CH, D))

    g = jnp.dot(xb, wg.astype(_BF), preferred_element_type=_F32)
    u = jnp.dot(xb, wu.astype(_BF), preferred_element_type=_F32)
    p = _silu(g) * u

    e_idx = (c - SC_CH) // PE
    lane = jax.lax.broadcasted_iota(jnp.int32, (T, E), 1)
    wcol = jnp.sum(jnp.where(lane == e_idx, w_ref[...], 0.0), axis=1,
                   keepdims=True)
    wcol = jnp.where(shared_phase, 1.0, wcol)
    pb = (p * wcol).astype(_BF)

    @pl.when(c == 0)
    def _():
        out_ref[...] = jnp.zeros((T, D), _F32)

    wdb = wd.astype(_BF)
    for h in range(4):
        lo, hi = h * (D // 4), (h + 1) * (D // 4)
        out_ref[:, lo:hi] += jnp.dot(pb, wdb[:, lo:hi],
                                     preferred_element_type=_F32)


def kernel(hidden_states, W_gate, Ws_gate, Ws_up, Ws_down, We_gate, We_up,
           We_down):
    B, S, Dm = hidden_states.shape
    x = hidden_states.reshape(T, D)
    xbf = x.astype(_BF)

    w = pl.pallas_call(
        _gate_body,
        in_specs=[
            pl.BlockSpec((T, D), lambda: (0, 0)),
            pl.BlockSpec((E, D), lambda: (0, 0)),
        ],
        out_specs=pl.BlockSpec((T, E), lambda: (0, 0)),
        out_shape=jax.ShapeDtypeStruct((T, E), _F32),
    )(x, W_gate)

    def _cr(c):
        return jnp.maximum(c - SC_CH, 0)

    out = pl.pallas_call(
        _moe_body,
        grid=(NC,),
        in_specs=[
            pl.BlockSpec((T, D), lambda c: (0, 0)),
            pl.BlockSpec((T, E), lambda c: (0, 0)),
            pl.BlockSpec((D, CH), lambda c: (0, jnp.minimum(c, SC_CH - 1))),
            pl.BlockSpec((D, CH), lambda c: (0, jnp.minimum(c, SC_CH - 1))),
            pl.BlockSpec((CH, D), lambda c: (jnp.minimum(c, SC_CH - 1), 0)),
            pl.BlockSpec((1, D, CH), lambda c: (_cr(c) // PE, 0, _cr(c) % PE)),
            pl.BlockSpec((1, D, CH), lambda c: (_cr(c) // PE, 0, _cr(c) % PE)),
            pl.BlockSpec((1, CH, D), lambda c: (_cr(c) // PE, _cr(c) % PE, 0)),
        ],
        out_specs=pl.BlockSpec((T, D), lambda c: (0, 0)),
        out_shape=jax.ShapeDtypeStruct((T, D), _F32),
    )(xbf, w, Ws_gate, Ws_up, Ws_down, We_gate, We_up, We_down)

    return out.reshape(B, S, Dm)


# speedup vs baseline: 1.7547x; 1.0026x over previous
import jax
import jax.numpy as jnp
from jax.experimental import pallas as pl
from jax.experimental.pallas import tpu as pltpu

E = 8
TOP_K = 2
D = 2048
D_FF = 1024
SHARED_FF = 2048
RSF = 2.5
T = 2048

CH = 256
SC_CH = SHARED_FF // CH
PE = D_FF // CH
NC = SC_CH + E * PE

_BF = jnp.bfloat16
_F32 = jnp.float32


def _silu(x):
    return x * jax.nn.sigmoid(x)


def _gate_body(x_ref, wgate_ref, w_ref):
    logits = jax.lax.dot_general(
        x_ref[...], wgate_ref[...], (((1,), (1,)), ((), ())),
        preferred_element_type=_F32)
    s = jax.nn.sigmoid(logits)
    lane = jax.lax.broadcasted_iota(jnp.int32, (T, E), 1)
    rank = jnp.zeros((T, E), jnp.int32)
    for j in range(E):
        sj = s[:, j:j + 1]
        rank += ((sj > s) | ((sj == s) & (j < lane))).astype(jnp.int32)
    w = jnp.where(rank < TOP_K, s, 0.0)
    denom = jnp.sum(w, axis=1, keepdims=True) + 1e-20
    w_ref[...] = w * (RSF / denom)


def _moe_body(xbf_ref, w_ref, sg_ref, su_ref, sd_ref,
              eg_ref, eu_ref, ed_ref, out_ref):
    c = pl.program_id(0)
    shared_phase = c < SC_CH
    xb = xbf_ref[...]

    wg = jnp.where(shared_phase, sg_ref[...], eg_ref[...].reshape(D, CH))
    wu = jnp.where(shared_phase, su_ref[...], eu_ref[...].reshape(D, CH))
    wd = jnp.where(shared_phase, sd_ref[...], ed_ref[...].reshape(CH, D))

    wgu = jnp.concatenate((wg.astype(_BF), wu.astype(_BF)), axis=1)
    gu = jnp.dot(xb, wgu, preferred_element_type=_F32)
    g, u = gu[:, :CH], gu[:, CH:]
    p = _silu(g) * u

    e_idx = (c - SC_CH) // PE
    lane = jax.lax.broadcasted_iota(jnp.int32, (T, E), 1)
    wcol = jnp.sum(jnp.where(lane == e_idx, w_ref[...], 0.0), axis=1,
                   keepdims=True)
    wcol = jnp.where(shared_phase, 1.0, wcol)
    pb = (p * wcol).astype(_BF)

    @pl.when(c == 0)
    def _():
        out_ref[...] = jnp.zeros((T, D), _F32)

    wdb = wd.astype(_BF)
    for h in range(4):
        lo, hi = h * (D // 4), (h + 1) * (D // 4)
        out_ref[:, lo:hi] += jnp.dot(pb, wdb[:, lo:hi],
                                     preferred_element_type=_F32)


def kernel(hidden_states, W_gate, Ws_gate, Ws_up, Ws_down, We_gate, We_up,
           We_down):
    B, S, Dm = hidden_states.shape
    x = hidden_states.reshape(T, D)
    xbf = x.astype(_BF)

    w = pl.pallas_call(
        _gate_body,
        in_specs=[
            pl.BlockSpec((T, D), lambda: (0, 0)),
            pl.BlockSpec((E, D), lambda: (0, 0)),
        ],
        out_specs=pl.BlockSpec((T, E), lambda: (0, 0)),
        out_shape=jax.ShapeDtypeStruct((T, E), _F32),
    )(x, W_gate)

    def _cr(c):
        return jnp.maximum(c - SC_CH, 0)

    out = pl.pallas_call(
        _moe_body,
        grid=(NC,),
        in_specs=[
            pl.BlockSpec((T, D), lambda c: (0, 0)),
            pl.BlockSpec((T, E), lambda c: (0, 0)),
            pl.BlockSpec((D, CH), lambda c: (0, jnp.minimum(c, SC_CH - 1))),
            pl.BlockSpec((D, CH), lambda c: (0, jnp.minimum(c, SC_CH - 1))),
            pl.BlockSpec((CH, D), lambda c: (jnp.minimum(c, SC_CH - 1), 0)),
            pl.BlockSpec((1, D, CH), lambda c: (_cr(c) // PE, 0, _cr(c) % PE)),
            pl.BlockSpec((1, D, CH), lambda c: (_cr(c) // PE, 0, _cr(c) % PE)),
            pl.BlockSpec((1, CH, D), lambda c: (_cr(c) // PE, _cr(c) % PE, 0)),
        ],
        out_specs=pl.BlockSpec((T, D), lambda c: (0, 0)),
        out_shape=jax.ShapeDtypeStruct((T, D), _F32),
    )(xbf, w, Ws_gate, Ws_up, Ws_down, We_gate, We_up, We_down)

    return out.reshape(B, S, Dm)
